# baseline (speedup 1.0000x reference)
import jax
import jax.numpy as jnp
from jax.experimental import pallas as pl
from jax.experimental.pallas import tpu as pltpu

_B, _H, _W, _C = 8, 224, 224, 384
_NROWS = _B * _H * _W
_ROW = 6 * _W + 6
_COL = 1
_TILE0 = (_ROW // 8) * 8
_NCHUNKS = 8


def _chunks():
    segs = [(0, _TILE0)]
    lo, hi = _TILE0 + 8, _NROWS
    n8 = (hi - lo) // 8
    per = (n8 // _NCHUNKS) * 8
    for i in range(_NCHUNKS):
        s = lo + i * per
        e = lo + (i + 1) * per if i < _NCHUNKS - 1 else hi
        segs.append((s, e - s))
    return segs


_SEGS = _chunks()


def _body(x_hbm, o_hbm, vbuf, sems):
    tile_in = pltpu.make_async_copy(
        x_hbm.at[pl.ds(_TILE0, 8), :], vbuf, sems.at[len(_SEGS)])
    tile_in.start()
    copies = []
    for i, (s, n) in enumerate(_SEGS):
        c = pltpu.make_async_copy(
            x_hbm.at[pl.ds(s, n), :], o_hbm.at[pl.ds(s, n), :], sems.at[i])
        c.start()
        copies.append(c)
    tile_in.wait()
    r = jax.lax.broadcasted_iota(jnp.int32, (8, _C), 0)
    c2 = jax.lax.broadcasted_iota(jnp.int32, (8, _C), 1)
    hit = (r == (_ROW - _TILE0)) & (c2 == _COL)
    vbuf[...] = jnp.where(hit, jnp.float32(1.0), vbuf[...])
    tile_out = pltpu.make_async_copy(
        vbuf, o_hbm.at[pl.ds(_TILE0, 8), :], sems.at[len(_SEGS)])
    tile_out.start()
    tile_out.wait()
    for c in copies:
        c.wait()


def kernel(x):
    xf = x.reshape(_NROWS, _C)
    out = pl.pallas_call(
        _body,
        in_specs=[pl.BlockSpec(memory_space=pl.ANY)],
        out_specs=pl.BlockSpec(memory_space=pl.ANY),
        out_shape=jax.ShapeDtypeStruct((_NROWS, _C), jnp.float32),
        scratch_shapes=[
            pltpu.VMEM((8, _C), jnp.float32),
            pltpu.SemaphoreType.DMA((len(_SEGS) + 1,)),
        ],
    )(xf)
    return out.reshape(_B, _H, _W, _C)


# speedup vs baseline: 47.8239x; 47.8239x over previous
import jax
import jax.numpy as jnp
from jax.experimental import pallas as pl
from jax.experimental.pallas import tpu as pltpu

_B, _H, _W, _C = 8, 224, 224, 384
_NROWS = _B * _H * _W
_ROW = 6 * _W + 6
_COL = 1
_BLOCK = 2048
_GRID = _NROWS // _BLOCK
_TBLK = _ROW // _BLOCK


def _body(x_ref, o_ref):
    o_ref[...] = x_ref[...]

    @pl.when(pl.program_id(0) == _TBLK)
    def _patch():
        r = jax.lax.broadcasted_iota(jnp.int32, (_BLOCK, _C), 0)
        c = jax.lax.broadcasted_iota(jnp.int32, (_BLOCK, _C), 1)
        hit = (r == (_ROW - _TBLK * _BLOCK)) & (c == _COL)
        o_ref[...] = jnp.where(hit, jnp.float32(1.0), o_ref[...])


def kernel(x):
    xf = x.reshape(_NROWS, _C)
    out = pl.pallas_call(
        _body,
        grid=(_GRID,),
        in_specs=[pl.BlockSpec((_BLOCK, _C), lambda i: (i, 0))],
        out_specs=pl.BlockSpec((_BLOCK, _C), lambda i: (i, 0)),
        out_shape=jax.ShapeDtypeStruct((_NROWS, _C), jnp.float32),
        compiler_params=pltpu.CompilerParams(
            dimension_semantics=("arbitrary",),
        ),
    )(xf)
    return out.reshape(_B, _H, _W, _C)


# speedup vs baseline: 49.2478x; 1.0298x over previous
import jax
import jax.numpy as jnp
from jax.experimental import pallas as pl
from jax.experimental.pallas import tpu as pltpu

_B, _H, _W, _C = 8, 224, 224, 384
_NROWS = _B * _H * _W
_ROW = 6 * _W + 6
_COL = 1
_BLOCK = 8192
_GRID = _NROWS // _BLOCK
_TBLK = _ROW // _BLOCK


def _body(x_ref, o_ref):
    o_ref[...] = x_ref[...]

    @pl.when(pl.program_id(0) == _TBLK)
    def _patch():
        r = jax.lax.broadcasted_iota(jnp.int32, (_BLOCK, _C), 0)
        c = jax.lax.broadcasted_iota(jnp.int32, (_BLOCK, _C), 1)
        hit = (r == (_ROW - _TBLK * _BLOCK)) & (c == _COL)
        o_ref[...] = jnp.where(hit, jnp.float32(1.0), o_ref[...])


def kernel(x):
    xf = x.reshape(_NROWS, _C)
    out = pl.pallas_call(
        _body,
        grid=(_GRID,),
        in_specs=[pl.BlockSpec((_BLOCK, _C), lambda i: (i, 0))],
        out_specs=pl.BlockSpec((_BLOCK, _C), lambda i: (i, 0)),
        out_shape=jax.ShapeDtypeStruct((_NROWS, _C), jnp.float32),
        compiler_params=pltpu.CompilerParams(
            dimension_semantics=("arbitrary",),
        ),
    )(xf)
    return out.reshape(_B, _H, _W, _C)


# speedup vs baseline: 49.2586x; 1.0002x over previous
import jax
import jax.numpy as jnp
from jax.experimental import pallas as pl
from jax.experimental.pallas import tpu as pltpu

_B, _H, _W, _C = 8, 224, 224, 384
_NROWS = _B * _H * _W
_ROW = 6 * _W + 6
_COL = 1
_RING = 8
_LAG = 3
_CHUNK = 4096
_N = _NROWS // _CHUNK

_TCHUNK = _ROW // _CHUNK
_TOFF = _ROW - _TCHUNK * _CHUNK
_TOFF8 = (_TOFF // 8) * 8


def _body(x_hbm, o_hbm, *rest):
    bufs = rest[:_RING]
    in_sems, out_sems = rest[_RING], rest[_RING + 1]
    in_copies = [None] * _N
    out_copies = [None] * _N
    out_waited = [False] * _N

    def start_in(i):
        b = i % _RING
        c = pltpu.make_async_copy(
            x_hbm.at[pl.ds(i * _CHUNK, _CHUNK), :], bufs[b], in_sems.at[b])
        c.start()
        in_copies[i] = c

    for i in range(min(_RING, _N)):
        start_in(i)
    for i in range(_N):
        b = i % _RING
        in_copies[i].wait()
        if i == _TCHUNK:
            r = jax.lax.broadcasted_iota(jnp.int32, (8, _C), 0)
            c2 = jax.lax.broadcasted_iota(jnp.int32, (8, _C), 1)
            hit = (r == (_TOFF - _TOFF8)) & (c2 == _COL)
            tile = bufs[b][pl.ds(_TOFF8, 8), :]
            bufs[b][pl.ds(_TOFF8, 8), :] = jnp.where(
                hit, jnp.float32(1.0), tile)
        oc = pltpu.make_async_copy(
            bufs[b], o_hbm.at[pl.ds(i * _CHUNK, _CHUNK), :], out_sems.at[b])
        oc.start()
        out_copies[i] = oc
        j = i - _LAG
        if j >= 0 and j + _RING < _N:
            out_copies[j].wait()
            out_waited[j] = True
            start_in(j + _RING)
    for i in range(_N):
        if not out_waited[i]:
            out_copies[i].wait()


def kernel(x):
    xf = x.reshape(_NROWS, _C)
    out = pl.pallas_call(
        _body,
        in_specs=[pl.BlockSpec(memory_space=pl.ANY)],
        out_specs=pl.BlockSpec(memory_space=pl.ANY),
        out_shape=jax.ShapeDtypeStruct((_NROWS, _C), jnp.float32),
        scratch_shapes=(
            [pltpu.VMEM((_CHUNK, _C), jnp.float32) for _ in range(_RING)]
            + [pltpu.SemaphoreType.DMA((_RING,)),
               pltpu.SemaphoreType.DMA((_RING,))]
        ),
    )(xf)
    return out.reshape(_B, _H, _W, _C)
